# baseline (speedup 1.0000x reference)
import jax
import jax.numpy as jnp
from jax.experimental import pallas as pl
from jax.experimental.pallas import tpu as pltpu


def _round_up(a, m):
    return ((a + m - 1) // m) * m


def _pad2(a, rows, cols, dtype):
    a = a.astype(dtype)
    if a.shape == (rows, cols):
        return a
    return jnp.zeros((rows, cols), dtype).at[: a.shape[0], : a.shape[1]].set(a)


def _gcn_body(x_ref, w_ref, adj_ref, b_ref, o_ref, sup_ref):
    i = pl.program_id(0)

    @pl.when(i == 0)
    def _compute_support():
        sup_ref[...] = jnp.dot(
            x_ref[...], w_ref[...], preferred_element_type=jnp.float32
        ).astype(sup_ref.dtype)

    a = adj_ref[...].astype(jnp.bfloat16)
    o_ref[...] = (
        jnp.dot(a, sup_ref[...], preferred_element_type=jnp.float32)
        + b_ref[...]
    )


def kernel(x, adj, weight, bias):
    x = jnp.squeeze(x)
    adj = jnp.squeeze(adj)
    N, Fin = x.shape
    Fout = weight.shape[1]
    if bias is None:
        bias = jnp.zeros((Fout,), jnp.float32)

    n_pad = _round_up(N, 512)
    fin_pad = _round_up(Fin, 128)
    fout_pad = _round_up(Fout, 128)

    x_p = _pad2(x, n_pad, fin_pad, jnp.float32)
    w_p = _pad2(weight, fin_pad, fout_pad, jnp.float32)
    adj_p = _pad2(adj, n_pad, n_pad, jnp.float32)
    b_p = _pad2(bias.reshape(1, Fout), 1, fout_pad, jnp.float32)

    tm = 256
    out_p = pl.pallas_call(
        _gcn_body,
        out_shape=jax.ShapeDtypeStruct((n_pad, fout_pad), jnp.float32),
        grid=(n_pad // tm,),
        in_specs=[
            pl.BlockSpec((n_pad, fin_pad), lambda i: (0, 0)),
            pl.BlockSpec((fin_pad, fout_pad), lambda i: (0, 0)),
            pl.BlockSpec((tm, n_pad), lambda i: (i, 0)),
            pl.BlockSpec((1, fout_pad), lambda i: (0, 0)),
        ],
        out_specs=pl.BlockSpec((tm, fout_pad), lambda i: (i, 0)),
        scratch_shapes=[pltpu.VMEM((n_pad, fout_pad), jnp.bfloat16)],
        compiler_params=pltpu.CompilerParams(
            dimension_semantics=("arbitrary",),
            vmem_limit_bytes=64 * 1024 * 1024,
        ),
        cost_estimate=pl.CostEstimate(
            flops=2 * n_pad * fout_pad * (n_pad + fin_pad),
            transcendentals=0,
            bytes_accessed=4 * n_pad * n_pad
            + 4 * n_pad * fin_pad
            + 4 * fin_pad * fout_pad
            + 4 * fout_pad
            + 4 * n_pad * fout_pad,
        ),
    )(x_p, w_p, adj_p, b_p)

    return out_p[:N, :Fout]


# speedup vs baseline: 1.0900x; 1.0900x over previous
import jax
import jax.numpy as jnp
from jax.experimental import pallas as pl
from jax.experimental.pallas import tpu as pltpu


def _round_up(a, m):
    return ((a + m - 1) // m) * m


def _pad2(a, rows, cols, dtype):
    a = a.astype(dtype)
    if a.shape == (rows, cols):
        return a
    return jnp.zeros((rows, cols), dtype).at[: a.shape[0], : a.shape[1]].set(a)


def _gcn_body(x_ref, w_ref, adj_ref, b_ref, o_ref, sup_ref):
    i = pl.program_id(0)

    @pl.when(i == 0)
    def _compute_support():
        sup_ref[...] = jnp.dot(
            x_ref[...], w_ref[...], preferred_element_type=jnp.float32
        ).astype(sup_ref.dtype)

    a = adj_ref[...].astype(jnp.bfloat16)
    o_ref[...] = (
        jnp.dot(a, sup_ref[...], preferred_element_type=jnp.float32)
        + b_ref[...]
    )


def kernel(x, adj, weight, bias):
    x = jnp.squeeze(x)
    adj = jnp.squeeze(adj)
    N, Fin = x.shape
    Fout = weight.shape[1]
    if bias is None:
        bias = jnp.zeros((Fout,), jnp.float32)

    n_pad = _round_up(N, 512)
    fin_pad = _round_up(Fin, 128)
    fout_pad = _round_up(Fout, 128)

    x_p = _pad2(x, n_pad, fin_pad, jnp.float32)
    w_p = _pad2(weight, fin_pad, fout_pad, jnp.float32)
    adj_p = _pad2(adj, n_pad, n_pad, jnp.float32)
    b_p = _pad2(bias.reshape(1, Fout), 1, fout_pad, jnp.float32)

    tm = 1024
    out_p = pl.pallas_call(
        _gcn_body,
        out_shape=jax.ShapeDtypeStruct((n_pad, fout_pad), jnp.float32),
        grid=(n_pad // tm,),
        in_specs=[
            pl.BlockSpec((n_pad, fin_pad), lambda i: (0, 0)),
            pl.BlockSpec((fin_pad, fout_pad), lambda i: (0, 0)),
            pl.BlockSpec((tm, n_pad), lambda i: (i, 0)),
            pl.BlockSpec((1, fout_pad), lambda i: (0, 0)),
        ],
        out_specs=pl.BlockSpec((tm, fout_pad), lambda i: (i, 0)),
        scratch_shapes=[pltpu.VMEM((n_pad, fout_pad), jnp.bfloat16)],
        compiler_params=pltpu.CompilerParams(
            dimension_semantics=("arbitrary",),
            vmem_limit_bytes=64 * 1024 * 1024,
        ),
        cost_estimate=pl.CostEstimate(
            flops=2 * n_pad * fout_pad * (n_pad + fin_pad),
            transcendentals=0,
            bytes_accessed=4 * n_pad * n_pad
            + 4 * n_pad * fin_pad
            + 4 * fin_pad * fout_pad
            + 4 * fout_pad
            + 4 * n_pad * fout_pad,
        ),
    )(x_p, w_p, adj_p, b_p)

    return out_p[:N, :Fout]


# speedup vs baseline: 1.1547x; 1.0594x over previous
import jax
import jax.numpy as jnp
from jax.experimental import pallas as pl
from jax.experimental.pallas import tpu as pltpu


def _round_up(a, m):
    return ((a + m - 1) // m) * m


def _pad2(a, rows, cols, dtype):
    a = a.astype(dtype)
    if a.shape == (rows, cols):
        return a
    return jnp.zeros((rows, cols), dtype).at[: a.shape[0], : a.shape[1]].set(a)


def _gcn_body(x_ref, w_ref, adj_ref, b_ref, o_ref, sup_ref):
    i = pl.program_id(0)

    @pl.when(i == 0)
    def _compute_support():
        sup_ref[...] = jnp.dot(
            x_ref[...], w_ref[...], preferred_element_type=jnp.float32
        ).astype(sup_ref.dtype)

    o_ref[...] = (
        jnp.dot(adj_ref[...], sup_ref[...],
                preferred_element_type=jnp.float32)
        + b_ref[...]
    )


def kernel(x, adj, weight, bias):
    x = jnp.squeeze(x)
    adj = jnp.squeeze(adj)
    N, Fin = x.shape
    Fout = weight.shape[1]
    if bias is None:
        bias = jnp.zeros((Fout,), jnp.float32)

    n_pad = _round_up(N, 512)
    fin_pad = _round_up(Fin, 128)
    fout_pad = _round_up(Fout, 128)

    x_p = _pad2(x, n_pad, fin_pad, jnp.float32)
    w_p = _pad2(weight, fin_pad, fout_pad, jnp.float32)
    adj_p = _pad2(adj, n_pad, n_pad, jnp.float32)
    b_p = _pad2(bias.reshape(1, Fout), 1, fout_pad, jnp.float32)

    tm = 512
    out_p = pl.pallas_call(
        _gcn_body,
        out_shape=jax.ShapeDtypeStruct((n_pad, fout_pad), jnp.float32),
        grid=(n_pad // tm,),
        in_specs=[
            pl.BlockSpec((n_pad, fin_pad), lambda i: (0, 0)),
            pl.BlockSpec((fin_pad, fout_pad), lambda i: (0, 0)),
            pl.BlockSpec((tm, n_pad), lambda i: (i, 0)),
            pl.BlockSpec((1, fout_pad), lambda i: (0, 0)),
        ],
        out_specs=pl.BlockSpec((tm, fout_pad), lambda i: (i, 0)),
        scratch_shapes=[pltpu.VMEM((n_pad, fout_pad), jnp.float32)],
        compiler_params=pltpu.CompilerParams(
            dimension_semantics=("arbitrary",),
            vmem_limit_bytes=64 * 1024 * 1024,
        ),
        cost_estimate=pl.CostEstimate(
            flops=2 * n_pad * fout_pad * (n_pad + fin_pad),
            transcendentals=0,
            bytes_accessed=4 * n_pad * n_pad
            + 4 * n_pad * fin_pad
            + 4 * fin_pad * fout_pad
            + 4 * fout_pad
            + 4 * n_pad * fout_pad,
        ),
    )(x_p, w_p, adj_p, b_p)

    return out_p[:N, :Fout]
